# Initial kernel scaffold; baseline (speedup 1.0000x reference)
import jax
import jax.numpy as jnp
from jax.experimental import pallas as pl


def kernel(x, edge_index, w1, b1, w2, b2, wg, bg):
    raise NotImplementedError("write your pallas kernel here")


# speedup vs baseline: 19.1644x; 19.1644x over previous
import functools

import jax
import jax.numpy as jnp
from jax import lax
from jax.experimental import pallas as pl
from jax.experimental.pallas import tpu as pltpu
from jax.experimental.pallas import tpu_sc as plsc

NC = 2
NS = 16
CH = 128


def _sc_edge_kernel(n_pad, d, cpw):
  rows_per_tile = n_pad // NS
  mesh = plsc.VectorSubcoreMesh(
      core_axis_name="c", subcore_axis_name="s", num_cores=NC,
      num_subcores=NS)

  @functools.partial(
      pl.kernel,
      mesh=mesh,
      out_type=(
          jax.ShapeDtypeStruct((n_pad, d), jnp.float32),
          jax.ShapeDtypeStruct((n_pad, d), jnp.float32),
      ),
      scratch_types=[
          pltpu.VMEM((cpw, CH), jnp.int32),
          pltpu.VMEM((cpw, CH), jnp.int32),
          pltpu.VMEM((cpw, CH), jnp.int32),
          pltpu.VMEM((CH,), jnp.float32),
          pltpu.VMEM((CH, d), jnp.float32),
          pltpu.SemaphoreType.DMA,
          pltpu.VMEM_SHARED((n_pad, d), jnp.float32),
      ],
  )
  def sc_edges(tgt_h, src_h, offs_h, x_h, zeros_h, ones_h, a_ref,
               p0_ref, p1_ref, tgt_v, src_v, offs_v, ones_v, rows_v, sem,
               agg1_sh):
    cid = lax.axis_index("c")
    sid = lax.axis_index("s")
    wid = cid * NS + sid
    stripe = pl.ds(sid * rows_per_tile, rows_per_tile)

    pltpu.sync_copy(zeros_h.at[stripe], agg1_sh.at[stripe])
    plsc.subcore_barrier()

    pltpu.sync_copy(ones_h, ones_v)
    rows = pl.ds(wid * cpw, cpw)
    pltpu.sync_copy(tgt_h.at[rows], tgt_v)
    pltpu.sync_copy(src_h.at[rows], src_v)
    pltpu.sync_copy(offs_h.at[rows], offs_v)

    def chunk(g, carry):
      pltpu.async_copy(x_h.at[tgt_v.at[g]], rows_v, sem).wait()
      pltpu.sync_copy(rows_v, agg1_sh.at[src_v.at[g]], add=True)
      pltpu.sync_copy(ones_v, a_ref.at[offs_v.at[g]])
      return carry

    lax.fori_loop(0, cpw, chunk, 0)

    plsc.subcore_barrier()

    @pl.when(cid == 0)
    def _():
      pltpu.sync_copy(agg1_sh.at[stripe], p0_ref.at[stripe])

    @pl.when(cid == 1)
    def _():
      pltpu.sync_copy(agg1_sh.at[stripe], p1_ref.at[stripe])

  return sc_edges


def _tc_fused_kernel(n_pad, d, bm, bn, bj):
  ni, nk, nj = n_pad // bm, n_pad // bn, n_pad // bj

  def body(a1_ref, a2_ref, x_ref, p0_ref, p1_ref, w1t_ref, b1_ref, w2t_ref,
           b2_ref, wg1_ref, wg2_ref, bg_ref, out_ref, c_acc, agg2_acc):
    i = pl.program_id(0)
    k = pl.program_id(1)
    j = pl.program_id(2)

    a1 = a1_ref[...].astype(jnp.bfloat16)
    a2 = a2_ref[...].astype(jnp.bfloat16)
    prev = jnp.where(j == 0, 0.0, c_acc[...])
    c_acc[...] = prev + jnp.dot(a1, a2, preferred_element_type=jnp.float32)

    @pl.when(j == nj - 1)
    def _():
      rows = i * bm + lax.broadcasted_iota(jnp.int32, (bm, bn), 0)
      cols = k * bn + lax.broadcasted_iota(jnp.int32, (bm, bn), 1)
      thr = jnp.where((c_acc[...] > 0.0) & (rows != cols), 1.0, 0.0)
      contrib = jnp.dot(thr, x_ref[...], preferred_element_type=jnp.float32)
      agg2_acc[...] = jnp.where(k == 0, 0.0, agg2_acc[...]) + contrib

      @pl.when(k == nk - 1)
      def _():
        agg1 = p0_ref[...] + p1_ref[...]
        z1 = jnp.dot(agg1, w1t_ref[...],
                     preferred_element_type=jnp.float32) + b1_ref[...]
        z2 = jnp.dot(agg2_acc[...], w2t_ref[...],
                     preferred_element_type=jnp.float32) + b2_ref[...]
        gate = jax.nn.sigmoid(
            jnp.dot(z1, wg1_ref[...], preferred_element_type=jnp.float32)
            + jnp.dot(z2, wg2_ref[...], preferred_element_type=jnp.float32)
            + bg_ref[...])
        out_ref[...] = gate * z1 + (1.0 - gate) * z2

  return pl.pallas_call(
      body,
      grid=(ni, nk, nj),
      in_specs=[
          pl.BlockSpec((bm, bj), lambda i, k, j: (i, j)),
          pl.BlockSpec((bj, bn), lambda i, k, j: (j, k)),
          pl.BlockSpec((bn, d), lambda i, k, j: (k, 0)),
          pl.BlockSpec((bm, d), lambda i, k, j: (i, 0)),
          pl.BlockSpec((bm, d), lambda i, k, j: (i, 0)),
          pl.BlockSpec((d, d), lambda i, k, j: (0, 0)),
          pl.BlockSpec((1, d), lambda i, k, j: (0, 0)),
          pl.BlockSpec((d, d), lambda i, k, j: (0, 0)),
          pl.BlockSpec((1, d), lambda i, k, j: (0, 0)),
          pl.BlockSpec((d, d), lambda i, k, j: (0, 0)),
          pl.BlockSpec((d, d), lambda i, k, j: (0, 0)),
          pl.BlockSpec((1, d), lambda i, k, j: (0, 0)),
      ],
      out_specs=pl.BlockSpec((bm, d), lambda i, k, j: (i, 0)),
      out_shape=jax.ShapeDtypeStruct((n_pad, d), jnp.float32),
      scratch_shapes=[
          pltpu.VMEM((bm, bn), jnp.float32),
          pltpu.VMEM((bm, d), jnp.float32),
      ],
      compiler_params=pltpu.CompilerParams(
          dimension_semantics=("parallel", "arbitrary", "arbitrary")),
  )


def kernel(x, edge_index, w1, b1, w2, b2, wg, bg):
  n, d = x.shape
  e = edge_index.shape[1]

  blk = 1024 if n >= 1024 else 128
  n_pad = ((n + blk - 1) // blk) * blk

  src = edge_index[0].astype(jnp.int32)
  tgt = edge_index[1].astype(jnp.int32)
  w = NC * NS
  cpw = -(-e // (w * CH))
  e_pad = w * cpw * CH
  pad = e_pad - e
  src_p = jnp.concatenate([src, jnp.full((pad,), n_pad - 1, jnp.int32)])
  tgt_p = jnp.concatenate([tgt, jnp.full((pad,), n_pad - 1, jnp.int32)])
  offs_p = src_p * n_pad + tgt_p
  src2 = src_p.reshape(w * cpw, CH)
  tgt2 = tgt_p.reshape(w * cpw, CH)
  offs2 = offs_p.reshape(w * cpw, CH)

  x_pad = jnp.zeros((n_pad, d), jnp.float32).at[:n].set(x)
  zeros2d = jnp.zeros((n_pad, d), jnp.float32)
  ones_c = jnp.ones((CH,), jnp.float32)

  a_ref = jax.new_ref(jnp.zeros((n_pad * n_pad,), jnp.float32))
  sc_edges = _sc_edge_kernel(n_pad, d, cpw)
  p0, p1 = sc_edges(tgt2, src2, offs2, x_pad, zeros2d, ones_c, a_ref)
  a = a_ref[...].reshape(n_pad, n_pad)

  bm = bn = bj = min(1024, n_pad)
  tc = _tc_fused_kernel(n_pad, d, bm, bn, bj)
  out = tc(a, a, x_pad, p0, p1,
           w1.T, b1.reshape(1, d), w2.T, b2.reshape(1, d),
           wg.T[:d], wg.T[d:], bg.reshape(1, d))
  return out[:n]


# speedup vs baseline: 22.2840x; 1.1628x over previous
import functools

import jax
import jax.numpy as jnp
from jax import lax
from jax.experimental import pallas as pl
from jax.experimental.pallas import tpu as pltpu
from jax.experimental.pallas import tpu_sc as plsc

NC = 2
NS = 16
CH = 128


def _sc_edge_kernel(n_pad, d, cpw):
  rows_per_tile = n_pad // NS
  mesh = plsc.VectorSubcoreMesh(
      core_axis_name="c", subcore_axis_name="s", num_cores=NC,
      num_subcores=NS)

  @functools.partial(
      pl.kernel,
      mesh=mesh,
      out_type=(
          jax.ShapeDtypeStruct((n_pad, d), jnp.float32),
          jax.ShapeDtypeStruct((n_pad, d), jnp.float32),
      ),
      scratch_types=[
          pltpu.VMEM((cpw, CH), jnp.int32),
          pltpu.VMEM((cpw, CH), jnp.int32),
          pltpu.VMEM((cpw, CH), jnp.int32),
          pltpu.VMEM((CH,), jnp.float32),
          pltpu.VMEM((CH, d), jnp.float32),
          pltpu.SemaphoreType.DMA,
          pltpu.VMEM_SHARED((n_pad, d), jnp.float32),
      ],
  )
  def sc_edges(tgt_h, src_h, offs_h, x_h, zeros_h, ones_h, a_ref,
               p0_ref, p1_ref, tgt_v, src_v, offs_v, ones_v, rows_v, sem,
               agg1_sh):
    cid = lax.axis_index("c")
    sid = lax.axis_index("s")
    wid = cid * NS + sid
    stripe = pl.ds(sid * rows_per_tile, rows_per_tile)

    pltpu.sync_copy(zeros_h.at[stripe], agg1_sh.at[stripe])
    plsc.subcore_barrier()

    pltpu.sync_copy(ones_h, ones_v)
    rows = pl.ds(wid * cpw, cpw)
    pltpu.sync_copy(tgt_h.at[rows], tgt_v)
    pltpu.sync_copy(src_h.at[rows], src_v)
    pltpu.sync_copy(offs_h.at[rows], offs_v)

    def chunk(g, carry):
      pltpu.async_copy(x_h.at[tgt_v.at[g]], rows_v, sem).wait()
      pltpu.sync_copy(rows_v, agg1_sh.at[src_v.at[g]], add=True)
      pltpu.sync_copy(ones_v, a_ref.at[offs_v.at[g]])
      return carry

    lax.fori_loop(0, cpw, chunk, 0)

    plsc.subcore_barrier()

    @pl.when(cid == 0)
    def _():
      pltpu.sync_copy(agg1_sh.at[stripe], p0_ref.at[stripe])

    @pl.when(cid == 1)
    def _():
      pltpu.sync_copy(agg1_sh.at[stripe], p1_ref.at[stripe])

  return sc_edges


def _tc_fused_kernel(n_pad, d, bm, bn, bj):
  ni, nk, nj = n_pad // bm, n_pad // bn, n_pad // bj

  def body(a1_ref, a2_ref, x_ref, p0_ref, p1_ref, w1t_ref, b1_ref, w2t_ref,
           b2_ref, wg1_ref, wg2_ref, bg_ref, out_ref, c_acc, agg2_acc):
    i = pl.program_id(0)
    k = pl.program_id(1)
    j = pl.program_id(2)

    prev = jnp.where(j == 0, 0.0, c_acc[...])
    c_acc[...] = prev + jnp.dot(a1_ref[...], a2_ref[...],
                                preferred_element_type=jnp.float32)

    @pl.when(j == nj - 1)
    def _():
      rows = i * bm + lax.broadcasted_iota(jnp.int32, (bm, bn), 0)
      cols = k * bn + lax.broadcasted_iota(jnp.int32, (bm, bn), 1)
      thr = jnp.where((c_acc[...] > 0.0) & (rows != cols), 1.0, 0.0)
      contrib = jnp.dot(thr, x_ref[...], preferred_element_type=jnp.float32)
      agg2_acc[...] = jnp.where(k == 0, 0.0, agg2_acc[...]) + contrib

      @pl.when(k == nk - 1)
      def _():
        agg1 = p0_ref[...] + p1_ref[...]
        z1 = jnp.dot(agg1, w1t_ref[...],
                     preferred_element_type=jnp.float32) + b1_ref[...]
        z2 = jnp.dot(agg2_acc[...], w2t_ref[...],
                     preferred_element_type=jnp.float32) + b2_ref[...]
        gate = jax.nn.sigmoid(
            jnp.dot(z1, wg1_ref[...], preferred_element_type=jnp.float32)
            + jnp.dot(z2, wg2_ref[...], preferred_element_type=jnp.float32)
            + bg_ref[...])
        out_ref[...] = gate * z1 + (1.0 - gate) * z2

  return pl.pallas_call(
      body,
      grid=(ni, nk, nj),
      in_specs=[
          pl.BlockSpec((bm, bj), lambda i, k, j: (i, j)),
          pl.BlockSpec((bj, bn), lambda i, k, j: (j, k)),
          pl.BlockSpec((bn, d), lambda i, k, j: (k, 0)),
          pl.BlockSpec((bm, d), lambda i, k, j: (i, 0)),
          pl.BlockSpec((bm, d), lambda i, k, j: (i, 0)),
          pl.BlockSpec((d, d), lambda i, k, j: (0, 0)),
          pl.BlockSpec((1, d), lambda i, k, j: (0, 0)),
          pl.BlockSpec((d, d), lambda i, k, j: (0, 0)),
          pl.BlockSpec((1, d), lambda i, k, j: (0, 0)),
          pl.BlockSpec((d, d), lambda i, k, j: (0, 0)),
          pl.BlockSpec((d, d), lambda i, k, j: (0, 0)),
          pl.BlockSpec((1, d), lambda i, k, j: (0, 0)),
      ],
      out_specs=pl.BlockSpec((bm, d), lambda i, k, j: (i, 0)),
      out_shape=jax.ShapeDtypeStruct((n_pad, d), jnp.float32),
      scratch_shapes=[
          pltpu.VMEM((bm, bn), jnp.float32),
          pltpu.VMEM((bm, d), jnp.float32),
      ],
      compiler_params=pltpu.CompilerParams(
          dimension_semantics=("parallel", "arbitrary", "arbitrary")),
  )


def kernel(x, edge_index, w1, b1, w2, b2, wg, bg):
  n, d = x.shape
  e = edge_index.shape[1]

  blk = 1024 if n >= 1024 else 128
  n_pad = ((n + blk - 1) // blk) * blk

  src = edge_index[0].astype(jnp.int32)
  tgt = edge_index[1].astype(jnp.int32)
  w = NC * NS
  cpw = -(-e // (w * CH))
  e_pad = w * cpw * CH
  pad = e_pad - e
  src_p = jnp.concatenate([src, jnp.full((pad,), n_pad - 1, jnp.int32)])
  tgt_p = jnp.concatenate([tgt, jnp.full((pad,), n_pad - 1, jnp.int32)])
  offs_p = src_p * n_pad + tgt_p
  src2 = src_p.reshape(w * cpw, CH)
  tgt2 = tgt_p.reshape(w * cpw, CH)
  offs2 = offs_p.reshape(w * cpw, CH)

  x_pad = jnp.zeros((n_pad, d), jnp.float32).at[:n].set(x)
  zeros2d = jnp.zeros((n_pad, d), jnp.float32)
  ones_c = jnp.ones((CH,), jnp.float32)

  a_ref = jax.new_ref(jnp.zeros((n_pad * n_pad,), jnp.float32))
  sc_edges = _sc_edge_kernel(n_pad, d, cpw)
  p0, p1 = sc_edges(tgt2, src2, offs2, x_pad, zeros2d, ones_c, a_ref)
  a = a_ref[...].reshape(n_pad, n_pad).astype(jnp.bfloat16)

  bm = bn = min(2048, n_pad)
  bj = min(1024, n_pad)
  tc = _tc_fused_kernel(n_pad, d, bm, bn, bj)
  out = tc(a, a, x_pad, p0, p1,
           w1.T, b1.reshape(1, d), w2.T, b2.reshape(1, d),
           wg.T[:d], wg.T[d:], bg.reshape(1, d))
  return out[:n]
